# baseline (speedup 1.0000x reference)
import jax
import jax.numpy as jnp
from jax.experimental import pallas as pl
from jax.experimental.pallas import tpu as pltpu

N = 4096
F = 64
LAM0 = 1.0
LAM1 = 1.0
LAM4 = 0.0
ALPHA = 1.0 / (1.0 + LAM4 + LAM0 + LAM1)

BLK1 = 512
G1 = N // BLK1

R0 = 128
P0_STEPS = N // R0
R1 = 512
P1_STEPS = N // R1
R2 = 1024
P2_STEPS = N // R2
TOTAL_STEPS = P0_STEPS + P1_STEPS + P2_STEPS


def _call1(ab_ref, ag_ref, xf_ref, xb_ref, s_ref, yhat_ref):
    s = LAM0 * ab_ref[...] + LAM1 * ag_ref[...]
    sb = s.astype(jnp.bfloat16)
    s_ref[...] = sb
    i = pl.program_id(0)
    yhat = (
        jnp.dot(sb, xf_ref[...].astype(jnp.bfloat16),
                preferred_element_type=jnp.float32)
        + xb_ref[...]
    )
    yhat_ref[pl.ds(i * BLK1, BLK1), :] = yhat.astype(jnp.bfloat16)


def _call2(db_ref, dg_ref, i_ref, s_ref, yhat1_ref, xf_ref, out_ref,
           qs_ref, y1_ref, y1b_ref, yhat2_ref):
    t = pl.program_id(0)

    @pl.when(t < P0_STEPS)
    def _():
        rows = pl.ds(t * R0, R0)
        qsb = (ALPHA / (LAM0 * db_ref[...] + LAM1 * dg_ref[...]
                        + i_ref[...])).astype(jnp.bfloat16)
        qs_ref[rows, :] = qsb
        y1 = (1.0 - ALPHA) * xf_ref[rows, :] + jnp.dot(
            qsb, yhat1_ref[...], preferred_element_type=jnp.float32)
        y1_ref[rows, :] = y1
        y1b_ref[rows, :] = y1.astype(jnp.bfloat16)

    @pl.when(jnp.logical_and(t >= P0_STEPS, t < P0_STEPS + P1_STEPS))
    def _():
        i = t - P0_STEPS
        rows = pl.ds(i * R1, R1)
        yhat2 = (
            jnp.dot(s_ref[...], y1b_ref[...],
                    preferred_element_type=jnp.float32)
            + xf_ref[rows, :]
        )
        yhat2_ref[rows, :] = yhat2.astype(jnp.bfloat16)

    @pl.when(t >= P0_STEPS + P1_STEPS)
    def _():
        i = t - (P0_STEPS + P1_STEPS)
        rows = pl.ds(i * R2, R2)
        out_ref[rows, :] = (1.0 - ALPHA) * y1_ref[rows, :] + jnp.dot(
            qs_ref[rows, :], yhat2_ref[...],
            preferred_element_type=jnp.float32)


def _d_map(t):
    return (jnp.where(t < P0_STEPS, t, N // R0 - 1), 0)


def _s_map(t):
    return (jnp.where(t < P0_STEPS, 0,
                      jnp.where(t < P0_STEPS + P1_STEPS, t - P0_STEPS,
                                P1_STEPS - 1)), 0)


def kernel(X, A_beta, A_gamma, D_beta, D_gamma, I_mat):
    f32 = jnp.float32
    bf16 = jnp.bfloat16

    s_bf16, yhat1b = pl.pallas_call(
        _call1,
        grid=(G1,),
        in_specs=[
            pl.BlockSpec((BLK1, N), lambda i: (i, 0)),
            pl.BlockSpec((BLK1, N), lambda i: (i, 0)),
            pl.BlockSpec((N, F), lambda i: (0, 0)),
            pl.BlockSpec((BLK1, F), lambda i: (i, 0)),
        ],
        out_specs=(
            pl.BlockSpec((BLK1, N), lambda i: (i, 0)),
            pl.BlockSpec((N, F), lambda i: (0, 0)),
        ),
        out_shape=(
            jax.ShapeDtypeStruct((N, N), bf16),
            jax.ShapeDtypeStruct((N, F), bf16),
        ),
        compiler_params=pltpu.CompilerParams(
            dimension_semantics=("arbitrary",)),
    )(A_beta, A_gamma, X, X)

    y2 = pl.pallas_call(
        _call2,
        grid=(TOTAL_STEPS,),
        in_specs=[
            pl.BlockSpec((R0, N), _d_map),
            pl.BlockSpec((R0, N), _d_map),
            pl.BlockSpec((R0, N), _d_map),
            pl.BlockSpec((R1, N), _s_map),
            pl.BlockSpec((N, F), lambda t: (0, 0)),
            pl.BlockSpec((N, F), lambda t: (0, 0)),
        ],
        out_specs=pl.BlockSpec((N, F), lambda t: (0, 0)),
        out_shape=jax.ShapeDtypeStruct((N, F), f32),
        scratch_shapes=[
            pltpu.VMEM((N, N), bf16),
            pltpu.VMEM((N, F), f32),
            pltpu.VMEM((N, F), bf16),
            pltpu.VMEM((N, F), bf16),
        ],
        compiler_params=pltpu.CompilerParams(
            dimension_semantics=("arbitrary",),
            vmem_limit_bytes=63 * 1024 * 1024),
    )(D_beta, D_gamma, I_mat, s_bf16, yhat1b, X)

    return y2
